# baseline (speedup 1.0000x reference)
import functools

import jax
import jax.numpy as jnp
from jax import lax
from jax.experimental import pallas as pl
from jax.experimental.pallas import tpu as pltpu
from jax.experimental.pallas import tpu_sc as plsc

N = 10000
E = 320000
H = 128
VOCAB = 10000

NC = 2
NS = 16
NW = NC * NS

CH_A = 80
NCH_A = N // CH_A
MAXJ_A = -(-NCH_A // NW)


def _gather_body(emb_hbm, tidx_hbm, h_hbm, idx_v, rows_v, sem):
    c = lax.axis_index("c")
    s = lax.axis_index("s")
    w = s * NC + c
    for j in range(MAXJ_A):
        cid = w + NW * j

        @pl.when(cid < NCH_A)
        def _():
            off = pl.multiple_of(cid * CH_A, 8)
            pltpu.sync_copy(tidx_hbm.at[pl.ds(off, CH_A)], idx_v)
            pltpu.async_copy(emb_hbm.at[idx_v], rows_v, sem).wait()
            pltpu.sync_copy(rows_v, h_hbm.at[pl.ds(off, CH_A)])


def _gather_h(emb, text_idx):
    mesh = plsc.VectorSubcoreMesh(core_axis_name="c", subcore_axis_name="s")
    return pl.kernel(
        _gather_body,
        out_type=jax.ShapeDtypeStruct((N, H), jnp.float32),
        mesh=mesh,
        scratch_types=[
            pltpu.VMEM((CH_A,), jnp.int32),
            pltpu.VMEM((CH_A, H), jnp.float32),
            pltpu.SemaphoreType.DMA,
        ],
    )(emb, text_idx)


RW = (-(-E // (NW * 128)) + 15) // 16 * 16
E_PAD = NW * RW * 128
RPS = 624
RPS_LAST = N - RPS * (NS - 1)
ACC_ROWS = N + 8


def _scatter_body(hw_hbm, src_hbm, dst_hbm, zeros_hbm, out_hbm,
                  sidx, didx, rows0, rows1, acc, sem0, sem1):
    c = lax.axis_index("c")
    s = lax.axis_index("s")
    w = s * NC + c
    row0 = pl.multiple_of(s * RPS, 8)
    @pl.when(s < NS - 1)
    def _():
        pltpu.sync_copy(zeros_hbm.at[pl.ds(0, RPS)], acc.at[pl.ds(row0, RPS)])

    @pl.when(s == NS - 1)
    def _():
        pltpu.sync_copy(zeros_hbm, acc.at[pl.ds(row0, RPS_LAST)])

    plsc.subcore_barrier()

    HRW = RW // 2
    for t in range(2):
        pltpu.sync_copy(src_hbm.at[w, pl.ds(t * HRW, HRW)], sidx)
        pltpu.sync_copy(dst_hbm.at[w, pl.ds(t * HRW, HRW)], didx)
        pltpu.async_copy(hw_hbm.at[sidx.at[0]], rows0, sem0)

        def body(p, carry):
            k = 2 * p
            pltpu.async_copy(hw_hbm.at[sidx.at[k + 1]], rows1, sem1)
            pltpu.make_async_copy(hw_hbm.at[sidx.at[k]], rows0, sem0).wait()
            pltpu.sync_copy(rows0, acc.at[didx.at[k]], add=True)

            @pl.when(p < HRW // 2 - 1)
            def _():
                pltpu.async_copy(hw_hbm.at[sidx.at[k + 2]], rows0, sem0)

            pltpu.make_async_copy(hw_hbm.at[sidx.at[k + 1]], rows1, sem1).wait()
            pltpu.sync_copy(rows1, acc.at[didx.at[k + 1]], add=True)
            return carry

        lax.fori_loop(0, HRW // 2, body, 0)
    plsc.subcore_barrier()

    @pl.when(s < NS - 1)
    def _():
        pltpu.sync_copy(acc.at[pl.ds(row0, RPS)], out_hbm.at[c, pl.ds(row0, RPS)])

    @pl.when(s == NS - 1)
    def _():
        pltpu.sync_copy(acc.at[pl.ds(row0, RPS_LAST)],
                        out_hbm.at[c, pl.ds(row0, RPS_LAST)])


def _edge_scatter(hw, src3d, dst3d, zeros):
    mesh = plsc.VectorSubcoreMesh(core_axis_name="c", subcore_axis_name="s")
    return pl.kernel(
        _scatter_body,
        out_type=jax.ShapeDtypeStruct((NC, N, H), jnp.float32),
        mesh=mesh,
        scratch_types=[
            pltpu.VMEM((RW // 2, 128), jnp.int32),
            pltpu.VMEM((RW // 2, 128), jnp.int32),
            pltpu.VMEM((128, H), jnp.float32),
            pltpu.VMEM((128, H), jnp.float32),
            pltpu.VMEM_SHARED((ACC_ROWS, H), jnp.float32),
            pltpu.SemaphoreType.DMA,
            pltpu.SemaphoreType.DMA,
        ],
    )(hw, src3d, dst3d, zeros)


RBLK = 1000


def _mm_body(h_ref, w_ref, o_ref):
    o_ref[...] = lax.dot_general(
        h_ref[...], w_ref[...], (((1,), (0,)), ((), ())),
        preferred_element_type=jnp.float32, precision=lax.Precision.HIGHEST)


def _edge_mm(h, w_edge):
    return pl.pallas_call(
        _mm_body,
        grid=(N // RBLK,),
        in_specs=[pl.BlockSpec((RBLK, H), lambda i: (i, 0)),
                  pl.BlockSpec((H, H), lambda i: (0, 0))],
        out_specs=pl.BlockSpec((RBLK, H), lambda i: (i, 0)),
        out_shape=jax.ShapeDtypeStruct((N, H), jnp.float32),
    )(h, w_edge)


def _dotf(a, b):
    return lax.dot_general(a, b, (((1,), (0,)), ((), ())),
                           preferred_element_type=jnp.float32,
                           precision=lax.Precision.HIGHEST)


def _gru_body(a0_ref, a1_ref, h_ref, wih_ref, whh_ref, bih_ref, bhh_ref,
              g1_ref, be1_ref, w1_ref, b1_ref, g2_ref, be2_ref, w2_ref, b2_ref,
              out_ref, agg_ref, m1_acc, m2_acc):
    i = pl.program_id(0)
    a = a0_ref[...] + a1_ref[...]
    h = h_ref[...]
    gi = _dotf(a, wih_ref[...]) + bih_ref[...]
    gh = _dotf(h, whh_ref[...]) + bhh_ref[...]
    r = jax.nn.sigmoid(gi[:, :H] + gh[:, :H])
    z = jax.nn.sigmoid(gi[:, H:2 * H] + gh[:, H:2 * H])
    n = jnp.tanh(gi[:, 2 * H:] + r * gh[:, 2 * H:])
    hn = (1.0 - z) * n + z * h
    bm1 = jnp.max(h, axis=0, keepdims=True)
    bm2 = jnp.max(hn, axis=0, keepdims=True)

    @pl.when(i == 0)
    def _():
        m1_acc[...] = bm1
        m2_acc[...] = bm2

    @pl.when(i > 0)
    def _():
        m1_acc[...] = jnp.maximum(m1_acc[...], bm1)
        m2_acc[...] = jnp.maximum(m2_acc[...], bm2)

    @pl.when(i == pl.num_programs(0) - 1)
    def _():
        agg = jnp.concatenate([m1_acc[...], m2_acc[...]], axis=1)
        mu = jnp.mean(agg, axis=-1, keepdims=True)
        var = jnp.mean((agg - mu) ** 2, axis=-1, keepdims=True)
        x = (agg - mu) * lax.rsqrt(var + 1e-5) * g1_ref[...] + be1_ref[...]
        x = jnp.maximum(_dotf(x, w1_ref[...]) + b1_ref[...], 0.0)
        mu2 = jnp.mean(x, axis=-1, keepdims=True)
        var2 = jnp.mean((x - mu2) ** 2, axis=-1, keepdims=True)
        x2 = (x - mu2) * lax.rsqrt(var2 + 1e-5) * g2_ref[...] + be2_ref[...]
        out_ref[...] = _dotf(x2, w2_ref[...]) + b2_ref[...]
        agg_ref[...] = agg


def _gru_head(a0, a1, h, w_ih, w_hh, b_ih, b_hh,
              ln1_g, ln1_b, w1, b1, ln2_g, ln2_b, w2, b2):
    full = lambda shape: pl.BlockSpec(shape, lambda i: tuple(0 for _ in shape))
    return pl.pallas_call(
        _gru_body,
        grid=(N // RBLK,),
        in_specs=[pl.BlockSpec((RBLK, H), lambda i: (i, 0)),
                  pl.BlockSpec((RBLK, H), lambda i: (i, 0)),
                  pl.BlockSpec((RBLK, H), lambda i: (i, 0)),
                  full((H, 3 * H)), full((H, 3 * H)),
                  full((1, 3 * H)), full((1, 3 * H)),
                  full((1, 2 * H)), full((1, 2 * H)),
                  full((2 * H, 64)), full((1, 64)),
                  full((1, 64)), full((1, 64)),
                  full((64, 1)), full((1, 1))],
        out_specs=[full((1, 1)), full((1, 2 * H))],
        out_shape=[jax.ShapeDtypeStruct((1, 1), jnp.float32),
                   jax.ShapeDtypeStruct((1, 2 * H), jnp.float32)],
        scratch_shapes=[pltpu.VMEM((1, H), jnp.float32),
                        pltpu.VMEM((1, H), jnp.float32)],
    )(a0, a1, h, w_ih, w_hh, b_ih, b_hh,
      ln1_g, ln1_b, w1, b1, ln2_g, ln2_b, w2, b2)


def kernel(text_idx, edge_index, flow, emb, W_edge, W_ih, W_hh, b_ih, b_hh,
           ln1_g, ln1_b, W1, b1, ln2_g, ln2_b, W2, b2):
    del flow
    text_idx = text_idx.astype(jnp.int32)
    npad = E_PAD - E
    src3d = jnp.concatenate(
        [edge_index[0].astype(jnp.int32),
         jnp.zeros((npad,), jnp.int32)]).reshape(NW, RW, 128)
    dst3d = jnp.concatenate(
        [edge_index[1].astype(jnp.int32),
         jnp.full((npad,), N, jnp.int32)]).reshape(NW, RW, 128)
    zeros = jnp.zeros((RPS_LAST, H), jnp.float32)

    h = _gather_h(emb, text_idx)
    hw = _edge_mm(h, W_edge)
    a_parts = _edge_scatter(hw, src3d, dst3d, zeros)

    out, agg = _gru_head(
        a_parts[0], a_parts[1], h, W_ih, W_hh,
        b_ih.reshape(1, 3 * H), b_hh.reshape(1, 3 * H),
        ln1_g.reshape(1, 2 * H), ln1_b.reshape(1, 2 * H),
        W1, b1.reshape(1, 64),
        ln2_g.reshape(1, 64), ln2_b.reshape(1, 64),
        W2, b2.reshape(1, 1))
    return (out, agg)


# speedup vs baseline: 3.2452x; 3.2452x over previous
import functools

import jax
import jax.numpy as jnp
from jax import lax
from jax.experimental import pallas as pl
from jax.experimental.pallas import tpu as pltpu
from jax.experimental.pallas import tpu_sc as plsc

N = 10000
E = 320000
H = 128
VOCAB = 10000

NC = 2
NS = 16
NW = NC * NS

CH_A = 80
NCH_A = N // CH_A
MAXJ_A = -(-NCH_A // NW)


def _gather_body(emb_hbm, tidx_hbm, h_hbm, idx_v, rows_v, sem):
    c = lax.axis_index("c")
    s = lax.axis_index("s")
    w = s * NC + c
    for j in range(MAXJ_A):
        cid = w + NW * j

        @pl.when(cid < NCH_A)
        def _():
            off = pl.multiple_of(cid * CH_A, 8)
            pltpu.sync_copy(tidx_hbm.at[pl.ds(off, CH_A)], idx_v)
            pltpu.async_copy(emb_hbm.at[idx_v], rows_v, sem).wait()
            pltpu.sync_copy(rows_v, h_hbm.at[pl.ds(off, CH_A)])


def _gather_h(emb, text_idx):
    mesh = plsc.VectorSubcoreMesh(core_axis_name="c", subcore_axis_name="s")
    return pl.kernel(
        _gather_body,
        out_type=jax.ShapeDtypeStruct((N, H), jnp.float32),
        mesh=mesh,
        scratch_types=[
            pltpu.VMEM((CH_A,), jnp.int32),
            pltpu.VMEM((CH_A, H), jnp.float32),
            pltpu.SemaphoreType.DMA,
        ],
    )(emb, text_idx)


RW = (-(-E // (NW * 128)) + 15) // 16 * 16
E_PAD = NW * RW * 128
RPS = 624
RPS_LAST = N - RPS * (NS - 1)
NDUMMY = 64
ACC_ROWS = N + NDUMMY


def _scatter_body(hw_hbm, src_hbm, dst_hbm, zeros_hbm, out_hbm,
                  sidx, didx, rows0, rows1, acc, sem0, sem1):
    c = lax.axis_index("c")
    s = lax.axis_index("s")
    w = s * NC + c
    row0 = pl.multiple_of(s * RPS, 8)
    @pl.when(s < NS - 1)
    def _():
        pltpu.sync_copy(zeros_hbm.at[pl.ds(0, RPS)], acc.at[pl.ds(row0, RPS)])

    @pl.when(s == NS - 1)
    def _():
        pltpu.sync_copy(zeros_hbm, acc.at[pl.ds(row0, RPS_LAST)])

    plsc.subcore_barrier()

    HRW = RW // 2
    for t in range(2):
        pltpu.sync_copy(src_hbm.at[w, pl.ds(t * HRW, HRW)], sidx)
        pltpu.sync_copy(dst_hbm.at[w, pl.ds(t * HRW, HRW)], didx)
        pltpu.async_copy(hw_hbm.at[sidx.at[0]], rows0, sem0)

        def body(p, carry):
            k = 2 * p
            pltpu.async_copy(hw_hbm.at[sidx.at[k + 1]], rows1, sem1)
            pltpu.make_async_copy(hw_hbm.at[sidx.at[k]], rows0, sem0).wait()
            pltpu.sync_copy(rows0, acc.at[didx.at[k]], add=True)

            @pl.when(p < HRW // 2 - 1)
            def _():
                pltpu.async_copy(hw_hbm.at[sidx.at[k + 2]], rows0, sem0)

            pltpu.make_async_copy(hw_hbm.at[sidx.at[k + 1]], rows1, sem1).wait()
            pltpu.sync_copy(rows1, acc.at[didx.at[k + 1]], add=True)
            return carry

        lax.fori_loop(0, HRW // 2, body, 0)
    plsc.subcore_barrier()

    @pl.when(s < NS - 1)
    def _():
        pltpu.sync_copy(acc.at[pl.ds(row0, RPS)], out_hbm.at[c, pl.ds(row0, RPS)])

    @pl.when(s == NS - 1)
    def _():
        pltpu.sync_copy(acc.at[pl.ds(row0, RPS_LAST)],
                        out_hbm.at[c, pl.ds(row0, RPS_LAST)])


def _edge_scatter(hw, src3d, dst3d, zeros):
    mesh = plsc.VectorSubcoreMesh(core_axis_name="c", subcore_axis_name="s")
    return pl.kernel(
        _scatter_body,
        out_type=jax.ShapeDtypeStruct((NC, N, H), jnp.float32),
        mesh=mesh,
        scratch_types=[
            pltpu.VMEM((RW // 2, 128), jnp.int32),
            pltpu.VMEM((RW // 2, 128), jnp.int32),
            pltpu.VMEM((128, H), jnp.float32),
            pltpu.VMEM((128, H), jnp.float32),
            pltpu.VMEM_SHARED((ACC_ROWS, H), jnp.float32),
            pltpu.SemaphoreType.DMA,
            pltpu.SemaphoreType.DMA,
        ],
    )(hw, src3d, dst3d, zeros)


RBLK = 1000


def _mm_body(h_ref, w_ref, o_ref):
    o_ref[...] = lax.dot_general(
        h_ref[...], w_ref[...], (((1,), (0,)), ((), ())),
        preferred_element_type=jnp.float32, precision=lax.Precision.HIGHEST)


def _edge_mm(h, w_edge):
    return pl.pallas_call(
        _mm_body,
        grid=(N // RBLK,),
        in_specs=[pl.BlockSpec((RBLK, H), lambda i: (i, 0)),
                  pl.BlockSpec((H, H), lambda i: (0, 0))],
        out_specs=pl.BlockSpec((RBLK, H), lambda i: (i, 0)),
        out_shape=jax.ShapeDtypeStruct((N, H), jnp.float32),
    )(h, w_edge)


def _dotf(a, b):
    return lax.dot_general(a, b, (((1,), (0,)), ((), ())),
                           preferred_element_type=jnp.float32,
                           precision=lax.Precision.HIGHEST)


def _gru_body(a0_ref, a1_ref, h_ref, wih_ref, whh_ref, bih_ref, bhh_ref,
              g1_ref, be1_ref, w1_ref, b1_ref, g2_ref, be2_ref, w2_ref, b2_ref,
              out_ref, agg_ref, m1_acc, m2_acc):
    i = pl.program_id(0)
    a = a0_ref[...] + a1_ref[...]
    h = h_ref[...]
    gi = _dotf(a, wih_ref[...]) + bih_ref[...]
    gh = _dotf(h, whh_ref[...]) + bhh_ref[...]
    r = jax.nn.sigmoid(gi[:, :H] + gh[:, :H])
    z = jax.nn.sigmoid(gi[:, H:2 * H] + gh[:, H:2 * H])
    n = jnp.tanh(gi[:, 2 * H:] + r * gh[:, 2 * H:])
    hn = (1.0 - z) * n + z * h
    bm1 = jnp.max(h, axis=0, keepdims=True)
    bm2 = jnp.max(hn, axis=0, keepdims=True)

    @pl.when(i == 0)
    def _():
        m1_acc[...] = bm1
        m2_acc[...] = bm2

    @pl.when(i > 0)
    def _():
        m1_acc[...] = jnp.maximum(m1_acc[...], bm1)
        m2_acc[...] = jnp.maximum(m2_acc[...], bm2)

    @pl.when(i == pl.num_programs(0) - 1)
    def _():
        agg = jnp.concatenate([m1_acc[...], m2_acc[...]], axis=1)
        mu = jnp.mean(agg, axis=-1, keepdims=True)
        var = jnp.mean((agg - mu) ** 2, axis=-1, keepdims=True)
        x = (agg - mu) * lax.rsqrt(var + 1e-5) * g1_ref[...] + be1_ref[...]
        x = jnp.maximum(_dotf(x, w1_ref[...]) + b1_ref[...], 0.0)
        mu2 = jnp.mean(x, axis=-1, keepdims=True)
        var2 = jnp.mean((x - mu2) ** 2, axis=-1, keepdims=True)
        x2 = (x - mu2) * lax.rsqrt(var2 + 1e-5) * g2_ref[...] + be2_ref[...]
        out_ref[...] = _dotf(x2, w2_ref[...]) + b2_ref[...]
        agg_ref[...] = agg


def _gru_head(a0, a1, h, w_ih, w_hh, b_ih, b_hh,
              ln1_g, ln1_b, w1, b1, ln2_g, ln2_b, w2, b2):
    full = lambda shape: pl.BlockSpec(shape, lambda i: tuple(0 for _ in shape))
    return pl.pallas_call(
        _gru_body,
        grid=(N // RBLK,),
        in_specs=[pl.BlockSpec((RBLK, H), lambda i: (i, 0)),
                  pl.BlockSpec((RBLK, H), lambda i: (i, 0)),
                  pl.BlockSpec((RBLK, H), lambda i: (i, 0)),
                  full((H, 3 * H)), full((H, 3 * H)),
                  full((1, 3 * H)), full((1, 3 * H)),
                  full((1, 2 * H)), full((1, 2 * H)),
                  full((2 * H, 64)), full((1, 64)),
                  full((1, 64)), full((1, 64)),
                  full((64, 1)), full((1, 1))],
        out_specs=[full((1, 1)), full((1, 2 * H))],
        out_shape=[jax.ShapeDtypeStruct((1, 1), jnp.float32),
                   jax.ShapeDtypeStruct((1, 2 * H), jnp.float32)],
        scratch_shapes=[pltpu.VMEM((1, H), jnp.float32),
                        pltpu.VMEM((1, H), jnp.float32)],
    )(a0, a1, h, w_ih, w_hh, b_ih, b_hh,
      ln1_g, ln1_b, w1, b1, ln2_g, ln2_b, w2, b2)


def kernel(text_idx, edge_index, flow, emb, W_edge, W_ih, W_hh, b_ih, b_hh,
           ln1_g, ln1_b, W1, b1, ln2_g, ln2_b, W2, b2):
    del flow
    text_idx = text_idx.astype(jnp.int32)
    npad = E_PAD - E
    spread = jnp.arange(npad, dtype=jnp.int32) % NDUMMY
    src3d = jnp.concatenate(
        [edge_index[0].astype(jnp.int32), spread]).reshape(NW, RW, 128)
    dst3d = jnp.concatenate(
        [edge_index[1].astype(jnp.int32), N + spread]).reshape(NW, RW, 128)
    zeros = jnp.zeros((RPS_LAST, H), jnp.float32)

    h = _gather_h(emb, text_idx)
    hw = _edge_mm(h, W_edge)
    a_parts = _edge_scatter(hw, src3d, dst3d, zeros)

    out, agg = _gru_head(
        a_parts[0], a_parts[1], h, W_ih, W_hh,
        b_ih.reshape(1, 3 * H), b_hh.reshape(1, 3 * H),
        ln1_g.reshape(1, 2 * H), ln1_b.reshape(1, 2 * H),
        W1, b1.reshape(1, 64),
        ln2_g.reshape(1, 64), ln2_b.reshape(1, 64),
        W2, b2.reshape(1, 1))
    return (out, agg)


# speedup vs baseline: 3.7736x; 1.1628x over previous
import functools

import jax
import jax.numpy as jnp
from jax import lax
from jax.experimental import pallas as pl
from jax.experimental.pallas import tpu as pltpu
from jax.experimental.pallas import tpu_sc as plsc

N = 10000
E = 320000
H = 128
VOCAB = 10000

NC = 2
NS = 16
NW = NC * NS

CH_A = 80
NCH_A = N // CH_A
MAXJ_A = -(-NCH_A // NW)


def _gather_body(emb_hbm, tidx_hbm, h_hbm, idx_v, rows_v, sem):
    c = lax.axis_index("c")
    s = lax.axis_index("s")
    w = s * NC + c
    for j in range(MAXJ_A):
        cid = w + NW * j

        @pl.when(cid < NCH_A)
        def _():
            off = pl.multiple_of(cid * CH_A, 8)
            pltpu.sync_copy(tidx_hbm.at[pl.ds(off, CH_A)], idx_v)
            pltpu.async_copy(emb_hbm.at[idx_v], rows_v, sem).wait()
            pltpu.sync_copy(rows_v, h_hbm.at[pl.ds(off, CH_A)])


def _gather_h(emb, text_idx):
    mesh = plsc.VectorSubcoreMesh(core_axis_name="c", subcore_axis_name="s")
    return pl.kernel(
        _gather_body,
        out_type=jax.ShapeDtypeStruct((N, H), jnp.float32),
        mesh=mesh,
        scratch_types=[
            pltpu.VMEM((CH_A,), jnp.int32),
            pltpu.VMEM((CH_A, H), jnp.float32),
            pltpu.SemaphoreType.DMA,
        ],
    )(emb, text_idx)


RW = (-(-E // (NW * 128)) + 15) // 16 * 16
E_PAD = NW * RW * 128
RPS = 624
RPS_LAST = N - RPS * (NS - 1)
NDUMMY = 64
ACC_ROWS = N + NDUMMY


def _scatter_body(hw_hbm, src_hbm, dst_hbm, zeros_hbm, out_hbm,
                  sidx, didx, rows0, rows1, acc, sem0, sem1):
    c = lax.axis_index("c")
    s = lax.axis_index("s")
    w = s * NC + c
    row0 = pl.multiple_of(s * RPS, 8)
    @pl.when(s < NS - 1)
    def _():
        pltpu.sync_copy(zeros_hbm.at[pl.ds(0, RPS)], acc.at[pl.ds(row0, RPS)])

    @pl.when(s == NS - 1)
    def _():
        pltpu.sync_copy(zeros_hbm, acc.at[pl.ds(row0, RPS_LAST)])

    plsc.subcore_barrier()

    HRW = RW // 2
    for t in range(2):
        pltpu.sync_copy(src_hbm.at[w, pl.ds(t * HRW, HRW)], sidx)
        pltpu.sync_copy(dst_hbm.at[w, pl.ds(t * HRW, HRW)], didx)
        pltpu.async_copy(hw_hbm.at[sidx.at[0]], rows0, sem0)

        def body(p, carry):
            k = 2 * p
            pltpu.async_copy(hw_hbm.at[sidx.at[k + 1]], rows1, sem1)
            pltpu.make_async_copy(hw_hbm.at[sidx.at[k]], rows0, sem0).wait()
            pltpu.sync_copy(rows0, acc.at[didx.at[k]], add=True)

            @pl.when(p < HRW // 2 - 1)
            def _():
                pltpu.async_copy(hw_hbm.at[sidx.at[k + 2]], rows0, sem0)

            pltpu.make_async_copy(hw_hbm.at[sidx.at[k + 1]], rows1, sem1).wait()
            pltpu.sync_copy(rows1, acc.at[didx.at[k + 1]], add=True)
            return carry

        lax.fori_loop(0, HRW // 2, body, 0)
    plsc.subcore_barrier()

    @pl.when(s < NS - 1)
    def _():
        pltpu.sync_copy(acc.at[pl.ds(row0, RPS)], out_hbm.at[c, pl.ds(row0, RPS)])

    @pl.when(s == NS - 1)
    def _():
        pltpu.sync_copy(acc.at[pl.ds(row0, RPS_LAST)],
                        out_hbm.at[c, pl.ds(row0, RPS_LAST)])


def _edge_scatter(hw, src3d, dst3d, zeros):
    mesh = plsc.VectorSubcoreMesh(core_axis_name="c", subcore_axis_name="s")
    return pl.kernel(
        _scatter_body,
        out_type=jax.ShapeDtypeStruct((NC, N, H), jnp.float32),
        mesh=mesh,
        scratch_types=[
            pltpu.VMEM((RW // 2, 128), jnp.int32),
            pltpu.VMEM((RW // 2, 128), jnp.int32),
            pltpu.VMEM((128, H), jnp.float32),
            pltpu.VMEM((128, H), jnp.float32),
            pltpu.VMEM_SHARED((ACC_ROWS, H), jnp.float32),
            pltpu.SemaphoreType.DMA,
            pltpu.SemaphoreType.DMA,
        ],
    )(hw, src3d, dst3d, zeros)


RBLK = 1000


def _mm_body(h_ref, w_ref, o_ref):
    o_ref[...] = lax.dot_general(
        h_ref[...], w_ref[...], (((1,), (0,)), ((), ())),
        preferred_element_type=jnp.float32)


def _edge_mm(h, w_edge):
    return pl.pallas_call(
        _mm_body,
        grid=(N // RBLK,),
        in_specs=[pl.BlockSpec((RBLK, H), lambda i: (i, 0)),
                  pl.BlockSpec((H, H), lambda i: (0, 0))],
        out_specs=pl.BlockSpec((RBLK, H), lambda i: (i, 0)),
        out_shape=jax.ShapeDtypeStruct((N, H), jnp.float32),
    )(h, w_edge)


def _dotf(a, b):
    return lax.dot_general(a, b, (((1,), (0,)), ((), ())),
                           preferred_element_type=jnp.float32)


def _gru_body(a0_ref, a1_ref, h_ref, wih_ref, whh_ref, bih_ref, bhh_ref,
              g1_ref, be1_ref, w1_ref, b1_ref, g2_ref, be2_ref, w2_ref, b2_ref,
              out_ref, agg_ref, m1_acc, m2_acc):
    i = pl.program_id(0)
    a = a0_ref[...] + a1_ref[...]
    h = h_ref[...]
    gi = _dotf(a, wih_ref[...]) + bih_ref[...]
    gh = _dotf(h, whh_ref[...]) + bhh_ref[...]
    r = jax.nn.sigmoid(gi[:, :H] + gh[:, :H])
    z = jax.nn.sigmoid(gi[:, H:2 * H] + gh[:, H:2 * H])
    n = jnp.tanh(gi[:, 2 * H:] + r * gh[:, 2 * H:])
    hn = (1.0 - z) * n + z * h
    bm1 = jnp.max(h, axis=0, keepdims=True)
    bm2 = jnp.max(hn, axis=0, keepdims=True)

    @pl.when(i == 0)
    def _():
        m1_acc[...] = bm1
        m2_acc[...] = bm2

    @pl.when(i > 0)
    def _():
        m1_acc[...] = jnp.maximum(m1_acc[...], bm1)
        m2_acc[...] = jnp.maximum(m2_acc[...], bm2)

    @pl.when(i == pl.num_programs(0) - 1)
    def _():
        agg = jnp.concatenate([m1_acc[...], m2_acc[...]], axis=1)
        mu = jnp.mean(agg, axis=-1, keepdims=True)
        var = jnp.mean((agg - mu) ** 2, axis=-1, keepdims=True)
        x = (agg - mu) / jnp.sqrt(var + 1e-5) * g1_ref[...] + be1_ref[...]
        x = jnp.maximum(_dotf(x, w1_ref[...]) + b1_ref[...], 0.0)
        mu2 = jnp.mean(x, axis=-1, keepdims=True)
        var2 = jnp.mean((x - mu2) ** 2, axis=-1, keepdims=True)
        x2 = (x - mu2) / jnp.sqrt(var2 + 1e-5) * g2_ref[...] + be2_ref[...]
        out_ref[...] = jnp.sum(x2 * w2_ref[...].reshape(1, 64), axis=-1,
                               keepdims=True) + b2_ref[...]
        agg_ref[...] = agg


def _gru_head(a0, a1, h, w_ih, w_hh, b_ih, b_hh,
              ln1_g, ln1_b, w1, b1, ln2_g, ln2_b, w2, b2):
    full = lambda shape: pl.BlockSpec(shape, lambda i: tuple(0 for _ in shape))
    return pl.pallas_call(
        _gru_body,
        grid=(N // RBLK,),
        in_specs=[pl.BlockSpec((RBLK, H), lambda i: (i, 0)),
                  pl.BlockSpec((RBLK, H), lambda i: (i, 0)),
                  pl.BlockSpec((RBLK, H), lambda i: (i, 0)),
                  full((H, 3 * H)), full((H, 3 * H)),
                  full((1, 3 * H)), full((1, 3 * H)),
                  full((1, 2 * H)), full((1, 2 * H)),
                  full((2 * H, 64)), full((1, 64)),
                  full((1, 64)), full((1, 64)),
                  full((64, 1)), full((1, 1))],
        out_specs=[full((1, 1)), full((1, 2 * H))],
        out_shape=[jax.ShapeDtypeStruct((1, 1), jnp.float32),
                   jax.ShapeDtypeStruct((1, 2 * H), jnp.float32)],
        scratch_shapes=[pltpu.VMEM((1, H), jnp.float32),
                        pltpu.VMEM((1, H), jnp.float32)],
    )(a0, a1, h, w_ih, w_hh, b_ih, b_hh,
      ln1_g, ln1_b, w1, b1, ln2_g, ln2_b, w2, b2)


def kernel(text_idx, edge_index, flow, emb, W_edge, W_ih, W_hh, b_ih, b_hh,
           ln1_g, ln1_b, W1, b1, ln2_g, ln2_b, W2, b2):
    del flow
    text_idx = text_idx.astype(jnp.int32)
    npad = E_PAD - E
    spread = jnp.arange(npad, dtype=jnp.int32) % NDUMMY
    src3d = jnp.concatenate(
        [edge_index[0].astype(jnp.int32), spread]).reshape(NW, RW, 128)
    dst3d = jnp.concatenate(
        [edge_index[1].astype(jnp.int32), N + spread]).reshape(NW, RW, 128)
    zeros = jnp.zeros((RPS_LAST, H), jnp.float32)

    h = _gather_h(emb, text_idx)
    hw = _edge_mm(h, W_edge)
    a_parts = _edge_scatter(hw, src3d, dst3d, zeros)

    out, agg = _gru_head(
        a_parts[0], a_parts[1], h, W_ih, W_hh,
        b_ih.reshape(1, 3 * H), b_hh.reshape(1, 3 * H),
        ln1_g.reshape(1, 2 * H), ln1_b.reshape(1, 2 * H),
        W1, b1.reshape(1, 64),
        ln2_g.reshape(1, 64), ln2_b.reshape(1, 64),
        W2, b2.reshape(1, 1))
    return (out, agg)
